# baseline (speedup 1.0000x reference)
import jax
import jax.numpy as jnp
from jax.experimental import pallas as pl
from jax.experimental.pallas import tpu as pltpu

_T, _B, _S = 4096, 16, 64
_K = 32
_C = _T // _K
_SUB = 32
_NSUB = _C // _SUB
_R = _K * _B
_LN2 = 0.6931471805599453
_LOG2E = 1.4426950408889634
_EXP_MASK = 0x7F800000


def _fwd_body(em_hbm, start_ref, stop_ref, trans_ref, out_ref,
              f_ref, b_ref, scf_ref, vbuf, dsem):
    i = pl.program_id(0)
    trans = trans_ref[...]
    tmax = jnp.max(trans)
    e2 = jnp.exp(trans - tmax)

    def vcopy(j, k):
        return pltpu.make_async_copy(
            em_hbm.at[pl.ds(k * _C + j * _SUB, _SUB)],
            vbuf.at[j, k],
            dsem.at[j],
        )

    def issue_slice(j):
        for k in range(_K):
            vcopy(j, k).start()

    def wait_slice(j):
        for k in range(_K):
            vcopy(j, k).wait()

    @pl.when(i == 0)
    def _():
        for j in (0, _NSUB - 1, 1, _NSUB - 2):
            issue_slice(j)
        wait_slice(0)
        wait_slice(_NSUB - 1)

    @pl.when(i == 1)
    def _():
        wait_slice(1)
        wait_slice(_NSUB - 2)

    def pos_fwd(F, comb):
        q = jax.lax.dot_general(
            F, e2, (((1,), (1,)), ((), ())), preferred_element_type=jnp.float32
        )
        return q * comb

    def pos_bwd(Bw, comb):
        return jax.lax.dot_general(
            Bw * comb, e2, (((1,), (0,)), ((), ())),
            preferred_element_type=jnp.float32,
        )

    def renorm_scale(x):
        c = jnp.max(x, axis=1, keepdims=True)
        eb = jax.lax.bitcast_convert_type(c, jnp.int32) & _EXP_MASK
        scale = jax.lax.bitcast_convert_type((254 << 23) - eb, jnp.float32)
        return scale, eb

    def emf_at(s):
        return vbuf[i, :, s].reshape(_R, _S)

    def emb_at(s):
        return vbuf[_NSUB - 1 - i, :, _SUB - 1 - s].reshape(_R, _S)

    def sweep(F, Bw, sf, scf, sb, start_pos):
        for s in range(start_pos, _SUB):
            comb_f = jnp.exp2(emf_at(s) * _LOG2E)
            comb_b = jnp.exp2(emb_at(s) * _LOG2E)
            if sf is not None:
                comb_f = comb_f * sf
                comb_b = comb_b * sb
                sf = sb = None
            F = pos_fwd(F, comb_f)
            Bw = pos_bwd(Bw, comb_b)
            if s % 4 == 3:
                sf, ebf = renorm_scale(F)
                sb, _ = renorm_scale(Bw)
                scf = scf + jax.lax.shift_right_arithmetic(ebf, 23) - 127
        f_ref[...] = F * sf
        b_ref[...] = Bw * sb
        scf_ref[...] = scf

    @pl.when(i == 0)
    def _():
        ones = jnp.ones((_R, _S), jnp.float32)
        F = pos_fwd(ones, jnp.exp2(emf_at(0) * _LOG2E))
        Bw = pos_bwd(ones, jnp.exp2(emb_at(0) * _LOG2E))
        p0 = jnp.exp(start_ref[0, :][None, :] + vbuf[0, 0, 0])
        row = jax.lax.broadcasted_iota(jnp.int32, (_R, _S), 0)
        F = jnp.where(row < _B, jnp.concatenate([p0] * _K, axis=0), F)
        sweep(F, Bw, None, jnp.zeros((_R, 1), jnp.int32), None, 1)

    @pl.when(i > 0)
    def _():
        sweep(f_ref[...], b_ref[...], None, scf_ref[...], None, 0)

    @pl.when(i == _NSUB - 1)
    def _():
        F = f_ref[...]
        Bw = b_ref[...]
        scf = scf_ref[...]
        dk = jnp.sum(F[: _R - _B] * Bw[_B:], axis=1, keepdims=True)
        sk = jnp.sum(Bw[_B:], axis=1, keepdims=True)
        V = jnp.log(dk) - jnp.log(sk)
        stop = stop_ref[0, :]
        smax = jnp.max(stop)
        w = jnp.sum(
            F[_R - _B:] * jnp.exp(stop - smax)[None, :], axis=1, keepdims=True
        )
        acc = jnp.log(w)
        for j in range(_K - 1):
            acc = acc + V[j * _B : (j + 1) * _B]
        scft = scf[: _B]
        for j in range(1, _K):
            scft = scft + scf[j * _B : (j + 1) * _B]
        out = acc + scft.astype(jnp.float32) * _LN2 + smax + (_T - 1) * tmax
        out_ref[...] = out.T


def kernel(emission_factors, start_factors, stop_factors, transition_factors):
    out = pl.pallas_call(
        _fwd_body,
        grid=(_NSUB,),
        in_specs=[
            pl.BlockSpec(memory_space=pl.ANY),
            pl.BlockSpec((1, _S), lambda i: (0, 0)),
            pl.BlockSpec((1, _S), lambda i: (0, 0)),
            pl.BlockSpec((_S, _S), lambda i: (0, 0)),
        ],
        out_specs=pl.BlockSpec((1, _B), lambda i: (0, 0)),
        out_shape=jax.ShapeDtypeStruct((1, _B), jnp.float32),
        scratch_shapes=[
            pltpu.VMEM((_R, _S), jnp.float32),
            pltpu.VMEM((_R, _S), jnp.float32),
            pltpu.VMEM((_R, 1), jnp.int32),
            pltpu.VMEM((_NSUB, _K, _SUB, _B, _S), jnp.float32),
            pltpu.SemaphoreType.DMA((_NSUB,)),
        ],
    )(
        emission_factors,
        start_factors.reshape(1, _S),
        stop_factors.reshape(1, _S),
        transition_factors,
    )
    return out.reshape(_B)


# speedup vs baseline: 1.0922x; 1.0922x over previous
import jax
import jax.numpy as jnp
from jax.experimental import pallas as pl
from jax.experimental.pallas import tpu as pltpu

_T, _B, _S = 4096, 16, 64
_K = 64
_C = _T // _K
_SUB = 16
_NSUB = _C // _SUB
_R = _K * _B
_LN2 = 0.6931471805599453
_LOG2E = 1.4426950408889634
_EXP_MASK = 0x7F800000


def _fwd_body(em_hbm, start_ref, stop_ref, trans_ref, out_ref,
              f_ref, b_ref, scf_ref, vbuf, dsem):
    i = pl.program_id(0)
    trans = trans_ref[...]
    tmax = jnp.max(trans)
    e2 = jnp.exp(trans - tmax)

    def vcopy(j, k):
        return pltpu.make_async_copy(
            em_hbm.at[pl.ds(k * _C + j * _SUB, _SUB)],
            vbuf.at[j, k],
            dsem.at[j],
        )

    def issue_slice(j):
        for k in range(_K):
            vcopy(j, k).start()

    def wait_slice(j):
        for k in range(_K):
            vcopy(j, k).wait()

    @pl.when(i == 0)
    def _():
        for j in (0, _NSUB - 1, 1, _NSUB - 2):
            issue_slice(j)
        wait_slice(0)
        wait_slice(_NSUB - 1)

    @pl.when(i == 1)
    def _():
        wait_slice(1)
        wait_slice(_NSUB - 2)

    def pos_fwd(F, comb):
        q = jax.lax.dot_general(
            F, e2, (((1,), (1,)), ((), ())), preferred_element_type=jnp.float32
        )
        return q * comb

    def pos_bwd(Bw, comb):
        return jax.lax.dot_general(
            Bw * comb, e2, (((1,), (0,)), ((), ())),
            preferred_element_type=jnp.float32,
        )

    def renorm_scale(x):
        c = jnp.max(x, axis=1, keepdims=True)
        eb = jax.lax.bitcast_convert_type(c, jnp.int32) & _EXP_MASK
        scale = jax.lax.bitcast_convert_type((254 << 23) - eb, jnp.float32)
        return scale, eb

    def emf_at(s):
        return vbuf[i, :, s].reshape(_R, _S)

    def emb_at(s):
        return vbuf[_NSUB - 1 - i, :, _SUB - 1 - s].reshape(_R, _S)

    def sweep(F, Bw, sf, scf, sb, start_pos):
        for s in range(start_pos, _SUB):
            comb_f = jnp.exp2(emf_at(s) * _LOG2E)
            comb_b = jnp.exp2(emb_at(s) * _LOG2E)
            if sf is not None:
                comb_f = comb_f * sf
                comb_b = comb_b * sb
                sf = sb = None
            F = pos_fwd(F, comb_f)
            Bw = pos_bwd(Bw, comb_b)
            if s % 4 == 3:
                sf, ebf = renorm_scale(F)
                sb, _ = renorm_scale(Bw)
                scf = scf + jax.lax.shift_right_arithmetic(ebf, 23) - 127
        f_ref[...] = F * sf
        b_ref[...] = Bw * sb
        scf_ref[...] = scf

    @pl.when(i == 0)
    def _():
        ones = jnp.ones((_R, _S), jnp.float32)
        F = pos_fwd(ones, jnp.exp2(emf_at(0) * _LOG2E))
        Bw = pos_bwd(ones, jnp.exp2(emb_at(0) * _LOG2E))
        p0 = jnp.exp(start_ref[0, :][None, :] + vbuf[0, 0, 0])
        row = jax.lax.broadcasted_iota(jnp.int32, (_R, _S), 0)
        F = jnp.where(row < _B, jnp.concatenate([p0] * _K, axis=0), F)
        sweep(F, Bw, None, jnp.zeros((_R, 1), jnp.int32), None, 1)

    @pl.when(i > 0)
    def _():
        sweep(f_ref[...], b_ref[...], None, scf_ref[...], None, 0)

    @pl.when(i == _NSUB - 1)
    def _():
        F = f_ref[...]
        Bw = b_ref[...]
        scf = scf_ref[...]
        dk = jnp.sum(F[: _R - _B] * Bw[_B:], axis=1, keepdims=True)
        sk = jnp.sum(Bw[_B:], axis=1, keepdims=True)
        V = jnp.log(dk) - jnp.log(sk)
        stop = stop_ref[0, :]
        smax = jnp.max(stop)
        w = jnp.sum(
            F[_R - _B:] * jnp.exp(stop - smax)[None, :], axis=1, keepdims=True
        )
        acc = jnp.log(w)
        for j in range(_K - 1):
            acc = acc + V[j * _B : (j + 1) * _B]
        scft = scf[: _B]
        for j in range(1, _K):
            scft = scft + scf[j * _B : (j + 1) * _B]
        out = acc + scft.astype(jnp.float32) * _LN2 + smax + (_T - 1) * tmax
        out_ref[...] = out.T


def kernel(emission_factors, start_factors, stop_factors, transition_factors):
    out = pl.pallas_call(
        _fwd_body,
        grid=(_NSUB,),
        in_specs=[
            pl.BlockSpec(memory_space=pl.ANY),
            pl.BlockSpec((1, _S), lambda i: (0, 0)),
            pl.BlockSpec((1, _S), lambda i: (0, 0)),
            pl.BlockSpec((_S, _S), lambda i: (0, 0)),
        ],
        out_specs=pl.BlockSpec((1, _B), lambda i: (0, 0)),
        out_shape=jax.ShapeDtypeStruct((1, _B), jnp.float32),
        scratch_shapes=[
            pltpu.VMEM((_R, _S), jnp.float32),
            pltpu.VMEM((_R, _S), jnp.float32),
            pltpu.VMEM((_R, 1), jnp.int32),
            pltpu.VMEM((_NSUB, _K, _SUB, _B, _S), jnp.float32),
            pltpu.SemaphoreType.DMA((_NSUB,)),
        ],
    )(
        emission_factors,
        start_factors.reshape(1, _S),
        stop_factors.reshape(1, _S),
        transition_factors,
    )
    return out.reshape(_B)
